# baseline (speedup 1.0000x reference)
import jax
import jax.numpy as jnp
from jax.experimental import pallas as pl
from jax.experimental.pallas import tpu as pltpu

_K = 64
_D = 16
_N_ITER = 4
_B = 1000


def _body(x_ref, c0_ref, cent_out, counts_out, cent_s, sums_s, counts_s):
    it = pl.program_id(0)
    ib = pl.program_id(1)
    nb = pl.num_programs(1)

    @pl.when(jnp.logical_and(it == 0, ib == 0))
    def _():
        cent_s[...] = c0_ref[...]

    @pl.when(ib == 0)
    def _():
        sums_s[...] = jnp.zeros_like(sums_s)
        counts_s[...] = jnp.zeros_like(counts_s)

    xb = x_ref[...]
    c = cent_s[...]
    c2 = jnp.sum(c * c, axis=1)
    xc = jax.lax.dot_general(
        xb, c, (((1,), (1,)), ((), ())), preferred_element_type=jnp.float32
    )
    dist = c2[None, :] - 2.0 * xc
    assign = jnp.argmin(dist, axis=1).astype(jnp.int32)
    onehot = (
        jax.lax.broadcasted_iota(jnp.int32, (xb.shape[0], _K), 1) == assign[:, None]
    ).astype(jnp.float32)

    sums_s[...] += jax.lax.dot_general(
        onehot, xb, (((0,), (0,)), ((), ())),
        preferred_element_type=jnp.float32,
        precision=jax.lax.Precision.HIGHEST,
    )
    counts_s[...] += jax.lax.dot_general(
        onehot, jnp.ones((xb.shape[0], 8), jnp.float32),
        (((0,), (0,)), ((), ())),
        preferred_element_type=jnp.float32,
    )

    @pl.when(ib == nb - 1)
    def _():
        cnt = counts_s[:, 0:1]
        newc = sums_s[...] / jnp.maximum(cnt, 1.0)
        cent_s[...] = newc

        @pl.when(it == _N_ITER - 1)
        def _():
            cent_out[...] = newc
            counts_out[...] = counts_s[...]


def kernel(x, centroids):
    n = x.shape[0]
    assert n % _B == 0
    nb = n // _B
    cent, counts8 = pl.pallas_call(
        _body,
        grid=(_N_ITER, nb),
        in_specs=[
            pl.BlockSpec((_B, _D), lambda it, ib: (ib, 0)),
            pl.BlockSpec((_K, _D), lambda it, ib: (0, 0)),
        ],
        out_specs=[
            pl.BlockSpec((_K, _D), lambda it, ib: (0, 0)),
            pl.BlockSpec((_K, 8), lambda it, ib: (0, 0)),
        ],
        out_shape=[
            jax.ShapeDtypeStruct((_K, _D), jnp.float32),
            jax.ShapeDtypeStruct((_K, 8), jnp.float32),
        ],
        scratch_shapes=[
            pltpu.VMEM((_K, _D), jnp.float32),
            pltpu.VMEM((_K, _D), jnp.float32),
            pltpu.VMEM((_K, 8), jnp.float32),
        ],
    )(x, centroids)
    return cent, counts8[:, 0]


# speedup vs baseline: 107.5403x; 107.5403x over previous
import jax
import jax.numpy as jnp
from jax.experimental import pallas as pl
from jax.experimental.pallas import tpu as pltpu

_K = 64
_D = 16
_N_ITER = 4
_BT = 8192


def _body(n_valid, x_ref, c0_ref, cent_out, counts_out, cent_s, sums_s):
    it = pl.program_id(0)
    ib = pl.program_id(1)
    nb = pl.num_programs(1)
    bt = x_ref.shape[1]

    @pl.when(jnp.logical_and(it == 0, ib == 0))
    def _():
        cent_s[...] = c0_ref[...]

    @pl.when(ib == 0)
    def _():
        sums_s[...] = jnp.zeros_like(sums_s)

    xb = x_ref[...]
    c = cent_s[...]
    h = 0.5 * jnp.sum(c * c, axis=1, keepdims=True)
    scores = jax.lax.dot_general(
        c, xb, (((1,), (0,)), ((), ())), preferred_element_type=jnp.float32
    )
    dist = h - scores
    m = jnp.min(dist, axis=0, keepdims=True)
    ii = jax.lax.broadcasted_iota(jnp.int32, (_K, bt), 0)
    assign = jnp.min(jnp.where(dist == m, ii, _K), axis=0, keepdims=True)
    col = ib * bt + jax.lax.broadcasted_iota(jnp.int32, (1, bt), 1)
    assign = jnp.where(col < n_valid, assign, -1)
    onehot = (ii == assign).astype(jnp.bfloat16)

    xa = jnp.concatenate([xb, jnp.ones((8, bt), jnp.float32)], axis=0)
    xh = xa.astype(jnp.bfloat16)
    xl = (xa - xh.astype(jnp.float32)).astype(jnp.bfloat16)
    sums_s[...] += jax.lax.dot_general(
        onehot, xh.T, (((1,), (0,)), ((), ())),
        preferred_element_type=jnp.float32,
    ) + jax.lax.dot_general(
        onehot, xl.T, (((1,), (0,)), ((), ())),
        preferred_element_type=jnp.float32,
    )

    @pl.when(ib == nb - 1)
    def _():
        cnt = sums_s[:, _D:_D + 1]
        newc = sums_s[:, :_D] / jnp.maximum(cnt, 1.0)
        cent_s[...] = newc

        @pl.when(it == _N_ITER - 1)
        def _():
            cent_out[...] = newc
            counts_out[...] = cnt


def kernel(x, centroids):
    import functools

    n = x.shape[0]
    nb = pl.cdiv(n, _BT)
    npad = nb * _BT - n
    xt = jnp.pad(x.T, ((0, 0), (0, npad)))
    cent, counts = pl.pallas_call(
        functools.partial(_body, n),
        grid=(_N_ITER, nb),
        in_specs=[
            pl.BlockSpec((_D, _BT), lambda it, ib: (0, ib)),
            pl.BlockSpec((_K, _D), lambda it, ib: (0, 0)),
        ],
        out_specs=[
            pl.BlockSpec((_K, _D), lambda it, ib: (0, 0)),
            pl.BlockSpec((_K, 1), lambda it, ib: (0, 0)),
        ],
        out_shape=[
            jax.ShapeDtypeStruct((_K, _D), jnp.float32),
            jax.ShapeDtypeStruct((_K, 1), jnp.float32),
        ],
        scratch_shapes=[
            pltpu.VMEM((_K, _D), jnp.float32),
            pltpu.VMEM((_K, _D + 8), jnp.float32),
        ],
    )(xt, centroids)
    return cent, counts[:, 0]
